# baseline (speedup 1.0000x reference)
import functools

import jax
import jax.numpy as jnp
from jax import lax
from jax.experimental import pallas as pl
from jax.experimental.pallas import tpu as pltpu
from jax.experimental.pallas import tpu_sc as plsc

K1 = 17
TN = 128
NC = 2
NS = 16
NW = NC * NS
L = 16


def _knn_kernel(pc_tile_ref, pc_all_ref, idx_ref):
    pc_t = pc_tile_ref[0]
    pc_a = pc_all_ref[0]
    n = pc_a.shape[1]
    tn = pc_t.shape[1]

    sq_t = jnp.sum(pc_t * pc_t, axis=0)
    sq_a = jnp.sum(pc_a * pc_a, axis=0)
    dot = lax.dot_general(pc_t, pc_a, (((0,), (0,)), ((), ())),
                          preferred_element_type=jnp.float32)
    work = sq_t[:, None] + sq_a[None, :] - 2.0 * dot

    iota = lax.broadcasted_iota(jnp.int32, (tn, n), 1)
    for k in range(K1):
        m = jnp.min(work, axis=1)
        am = jnp.min(jnp.where(work == m[:, None], iota, n), axis=1)
        work = jnp.where(iota == am[:, None], jnp.float32(jnp.inf), work)
        idx_ref[0, k, :] = am


def _sc_edge_body(n, b_total, pc_hbm, idx_hbm, out_hbm, x_v, y_v, z_v,
                  ix_v, dx_v, dy_v, dz_v):
    pairs = b_total * K1
    trips = (pairs + NW - 1) // NW
    wid = lax.axis_index("s") * NC + lax.axis_index("c")

    def pair_body(t, carry):
        p = wid + NW * t

        @pl.when(p < pairs)
        def _():
            b = p // K1
            k = p - b * K1
            pc_base = b * (3 * n)
            pltpu.sync_copy(pc_hbm.at[pl.ds(pc_base, n)], x_v)
            pltpu.sync_copy(pc_hbm.at[pl.ds(pc_base + n, n)], y_v)
            pltpu.sync_copy(pc_hbm.at[pl.ds(pc_base + 2 * n, n)], z_v)
            pltpu.sync_copy(idx_hbm.at[pl.ds(p * n, n)], ix_v)

            def chunk(i, c2):
                sl = pl.ds(i * L, L)
                jv = ix_v[sl]
                dx_v[sl] = plsc.load_gather(x_v, [jv]) - x_v[sl]
                dy_v[sl] = plsc.load_gather(y_v, [jv]) - y_v[sl]
                dz_v[sl] = plsc.load_gather(z_v, [jv]) - z_v[sl]
                return c2

            lax.fori_loop(0, n // L, chunk, 0)
            out_base = (b * 6 * K1 + k) * n
            kn = K1 * n
            pltpu.sync_copy(x_v, out_hbm.at[pl.ds(out_base, n)])
            pltpu.sync_copy(y_v, out_hbm.at[pl.ds(out_base + kn, n)])
            pltpu.sync_copy(z_v, out_hbm.at[pl.ds(out_base + 2 * kn, n)])
            pltpu.sync_copy(dx_v, out_hbm.at[pl.ds(out_base + 3 * kn, n)])
            pltpu.sync_copy(dy_v, out_hbm.at[pl.ds(out_base + 4 * kn, n)])
            pltpu.sync_copy(dz_v, out_hbm.at[pl.ds(out_base + 5 * kn, n)])

        return carry

    lax.fori_loop(0, trips, pair_body, 0)


def _half(point_cloud):
    B, D, N = point_cloud.shape
    idx = pl.pallas_call(
        _knn_kernel,
        grid=(B, N // TN),
        in_specs=[
            pl.BlockSpec((1, D, TN), lambda b, j: (b, 0, j)),
            pl.BlockSpec((1, D, N), lambda b, j: (b, 0, 0)),
        ],
        out_specs=pl.BlockSpec((1, K1, TN), lambda b, j: (b, 0, j)),
        out_shape=jax.ShapeDtypeStruct((B, K1, N), jnp.int32),
        compiler_params=pltpu.CompilerParams(
            dimension_semantics=("parallel", "parallel")),
    )(point_cloud, point_cloud)

    mesh = plsc.VectorSubcoreMesh(core_axis_name="c", subcore_axis_name="s")
    sc_edge = pl.kernel(
        functools.partial(_sc_edge_body, N, B),
        mesh=mesh,
        compiler_params=pltpu.CompilerParams(needs_layout_passes=False),
        out_type=jax.ShapeDtypeStruct((B * 2 * D * K1 * N,), jnp.float32),
        scratch_types=[
            pltpu.VMEM((N,), jnp.float32),
            pltpu.VMEM((N,), jnp.float32),
            pltpu.VMEM((N,), jnp.float32),
            pltpu.VMEM((N,), jnp.int32),
            pltpu.VMEM((N,), jnp.float32),
            pltpu.VMEM((N,), jnp.float32),
            pltpu.VMEM((N,), jnp.float32),
        ],
    )
    ef_flat = sc_edge(point_cloud.reshape(-1), idx.reshape(-1))
    return ef_flat.reshape(B, 2 * D, K1, N), idx


def kernel(point_cloud):
    B = point_cloud.shape[0]
    h = B // 2
    ef0, idx0 = _half(point_cloud[:h])
    ef1, idx1 = _half(point_cloud[h:])
    return (jnp.concatenate([ef0, ef1], axis=0),
            jnp.concatenate([idx0, idx1], axis=0))


# speedup vs baseline: 1.3643x; 1.3643x over previous
import functools

import jax
import jax.numpy as jnp
from jax import lax
from jax.experimental import pallas as pl
from jax.experimental.pallas import tpu as pltpu
from jax.experimental.pallas import tpu_sc as plsc

K1 = 17
TN = 128
NC = 2
NS = 16
NW = NC * NS
L = 16


def _knn_kernel(pc_tile_ref, pc_all_ref, idx_ref):
    pc_t = pc_tile_ref[0]
    pc_a = pc_all_ref[0]
    n = pc_a.shape[1]
    tn = pc_t.shape[1]

    sq_t = jnp.sum(pc_t * pc_t, axis=0)
    sq_a = jnp.sum(pc_a * pc_a, axis=0)
    dot = lax.dot_general(pc_t, pc_a, (((0,), (0,)), ((), ())),
                          preferred_element_type=jnp.float32)
    work = sq_t[:, None] + sq_a[None, :] - 2.0 * dot

    iota = lax.broadcasted_iota(jnp.int32, (tn, n), 1)
    for k in range(K1):
        am = jnp.argmin(work, axis=1).astype(jnp.int32)
        work = jnp.where(iota == am[:, None], jnp.float32(jnp.inf), work)
        idx_ref[0, k, :] = am


def _sc_edge_body(n, b_total, pc_hbm, idx_hbm, out_hbm, x_v, y_v, z_v,
                  ix_v, dx_v, dy_v, dz_v):
    pairs = b_total * K1
    trips = (pairs + NW - 1) // NW
    wid = lax.axis_index("s") * NC + lax.axis_index("c")

    def pair_body(t, carry):
        p = wid + NW * t

        @pl.when(p < pairs)
        def _():
            b = p // K1
            k = p - b * K1
            pc_base = b * (3 * n)
            pltpu.sync_copy(pc_hbm.at[pl.ds(pc_base, n)], x_v)
            pltpu.sync_copy(pc_hbm.at[pl.ds(pc_base + n, n)], y_v)
            pltpu.sync_copy(pc_hbm.at[pl.ds(pc_base + 2 * n, n)], z_v)
            pltpu.sync_copy(idx_hbm.at[pl.ds(p * n, n)], ix_v)

            def chunk(i, c2):
                sl = pl.ds(i * L, L)
                jv = ix_v[sl]
                dx_v[sl] = plsc.load_gather(x_v, [jv]) - x_v[sl]
                dy_v[sl] = plsc.load_gather(y_v, [jv]) - y_v[sl]
                dz_v[sl] = plsc.load_gather(z_v, [jv]) - z_v[sl]
                return c2

            lax.fori_loop(0, n // L, chunk, 0)
            out_base = (b * 6 * K1 + k) * n
            kn = K1 * n
            pltpu.sync_copy(x_v, out_hbm.at[pl.ds(out_base, n)])
            pltpu.sync_copy(y_v, out_hbm.at[pl.ds(out_base + kn, n)])
            pltpu.sync_copy(z_v, out_hbm.at[pl.ds(out_base + 2 * kn, n)])
            pltpu.sync_copy(dx_v, out_hbm.at[pl.ds(out_base + 3 * kn, n)])
            pltpu.sync_copy(dy_v, out_hbm.at[pl.ds(out_base + 4 * kn, n)])
            pltpu.sync_copy(dz_v, out_hbm.at[pl.ds(out_base + 5 * kn, n)])

        return carry

    lax.fori_loop(0, trips, pair_body, 0)


def _half(point_cloud):
    B, D, N = point_cloud.shape
    idx = pl.pallas_call(
        _knn_kernel,
        grid=(B, N // TN),
        in_specs=[
            pl.BlockSpec((1, D, TN), lambda b, j: (b, 0, j)),
            pl.BlockSpec((1, D, N), lambda b, j: (b, 0, 0)),
        ],
        out_specs=pl.BlockSpec((1, K1, TN), lambda b, j: (b, 0, j)),
        out_shape=jax.ShapeDtypeStruct((B, K1, N), jnp.int32),
        compiler_params=pltpu.CompilerParams(
            dimension_semantics=("parallel", "parallel")),
    )(point_cloud, point_cloud)

    mesh = plsc.VectorSubcoreMesh(core_axis_name="c", subcore_axis_name="s")
    sc_edge = pl.kernel(
        functools.partial(_sc_edge_body, N, B),
        mesh=mesh,
        compiler_params=pltpu.CompilerParams(needs_layout_passes=False),
        out_type=jax.ShapeDtypeStruct((B * 2 * D * K1 * N,), jnp.float32),
        scratch_types=[
            pltpu.VMEM((N,), jnp.float32),
            pltpu.VMEM((N,), jnp.float32),
            pltpu.VMEM((N,), jnp.float32),
            pltpu.VMEM((N,), jnp.int32),
            pltpu.VMEM((N,), jnp.float32),
            pltpu.VMEM((N,), jnp.float32),
            pltpu.VMEM((N,), jnp.float32),
        ],
    )
    ef_flat = sc_edge(point_cloud.reshape(-1), idx.reshape(-1))
    return ef_flat.reshape(B, 2 * D, K1, N), idx


def kernel(point_cloud):
    B = point_cloud.shape[0]
    step = 2
    efs, idxs = [], []
    for s in range(0, B, step):
        ef_c, idx_c = _half(point_cloud[s:s + step])
        efs.append(ef_c)
        idxs.append(idx_c)
    return (jnp.concatenate(efs, axis=0), jnp.concatenate(idxs, axis=0))
